# Initial kernel scaffold; baseline (speedup 1.0000x reference)
import jax
import jax.numpy as jnp
from jax.experimental import pallas as pl


def kernel(x, edge_index, edge_attr, W_enc, b_enc, W_dec, b_dec, W_msg, W_edge, b_msg, W_self):
    raise NotImplementedError("write your pallas kernel here")


# speedup vs baseline: 3.9991x; 3.9991x over previous
import functools

import jax
import jax.numpy as jnp
from jax import lax
from jax.experimental import pallas as pl
from jax.experimental.pallas import tpu as pltpu
from jax.experimental.pallas import tpu_sc as plsc

N = 10000
D = 128
E = 320000
NC = 2
NS = 16
NW = NC * NS
BLK = 128
EPW = 10240
E_PAD = NW * EPW
N_PAD = 10112
RPT = N_PAD // NS
ROWS_F32 = (D // 16)
_CHUNKS = [BLK] * (RPT // BLK) + ([RPT % BLK] if RPT % BLK else [])


def _prep_body(ea_ref, we_ref, wez_ref, b_ref, c0_ref, cz_ref):
    ea = ea_ref[...]
    b = b_ref[...]
    c0_ref[...] = jnp.dot(ea, we_ref[...], preferred_element_type=jnp.float32) + b
    cz_ref[...] = jnp.dot(ea, wez_ref[...], preferred_element_type=jnp.float32) + b


def _enc_body(x_ref, wenc_ref, benc_ref, wmsg_ref, h_ref, u_ref):
    h = jnp.dot(x_ref[...], wenc_ref[...], preferred_element_type=jnp.float32)
    h = h + benc_ref[...]
    h_ref[...] = h
    u_ref[...] = jnp.dot(h, wmsg_ref[...], preferred_element_type=jnp.float32)


def _mid_body(h_ref, agg0_ref, agg1_ref, wself_ref, wmsg_ref, wz_ref, bz_ref,
              h_out, u_out, z_out):
    h = jnp.dot(h_ref[...], wself_ref[...], preferred_element_type=jnp.float32)
    h = h + agg0_ref[...] + agg1_ref[...]
    z = jnp.dot(h, wz_ref[...], preferred_element_type=jnp.float32) + bz_ref[...]
    h_out[...] = h
    z_out[...] = z
    u_out[...] = jnp.dot(h, wmsg_ref[...], preferred_element_type=jnp.float32) + z


def _fin_body(h_ref, agg0_ref, agg1_ref, wself_ref, wdec_ref, bdec_ref, out_ref):
    h = jnp.dot(h_ref[...], wself_ref[...], preferred_element_type=jnp.float32)
    h = h + agg0_ref[...] + agg1_ref[...]
    out_ref[...] = jnp.dot(h, wdec_ref[...], preferred_element_type=jnp.float32)
    out_ref[...] += bdec_ref[...]


_W_SPEC = pl.BlockSpec((16, D), lambda i: (0, 0))
_WD_SPEC = pl.BlockSpec((D, D), lambda i: (0, 0))
_B_SPEC = pl.BlockSpec((1, D), lambda i: (0, 0))

_NBLK_ROWS = 2000
_ROW_SPEC = pl.BlockSpec((_NBLK_ROWS, D), lambda i: (i, 0))

_EBLK = 4096
_prep_call = pl.pallas_call(
    _prep_body,
    grid=(E_PAD // _EBLK,),
    in_specs=[pl.BlockSpec((_EBLK, 16), lambda i: (i, 0)), _W_SPEC, _W_SPEC,
              _B_SPEC],
    out_specs=[pl.BlockSpec((_EBLK, D), lambda i: (i, 0))] * 2,
    out_shape=[jax.ShapeDtypeStruct((E_PAD, D), jnp.float32)] * 2,
)

_enc_call = pl.pallas_call(
    _enc_body,
    grid=(N // _NBLK_ROWS,),
    in_specs=[_ROW_SPEC, _WD_SPEC, _B_SPEC, _WD_SPEC],
    out_specs=[_ROW_SPEC, _ROW_SPEC],
    out_shape=[jax.ShapeDtypeStruct((N, D), jnp.float32)] * 2,
)

_mid_call = pl.pallas_call(
    _mid_body,
    grid=(N // _NBLK_ROWS,),
    in_specs=[_ROW_SPEC, _ROW_SPEC, _ROW_SPEC, _WD_SPEC, _WD_SPEC, _WD_SPEC,
              _B_SPEC],
    out_specs=[_ROW_SPEC, _ROW_SPEC, _ROW_SPEC],
    out_shape=[jax.ShapeDtypeStruct((N, D), jnp.float32)] * 3,
)

_fin_call = pl.pallas_call(
    _fin_body,
    grid=(N // _NBLK_ROWS,),
    in_specs=[_ROW_SPEC, _ROW_SPEC, _ROW_SPEC, _WD_SPEC,
              pl.BlockSpec((D, 4), lambda i: (0, 0)),
              pl.BlockSpec((1, 4), lambda i: (0, 0))],
    out_specs=pl.BlockSpec((_NBLK_ROWS, 4), lambda i: (i, 0)),
    out_shape=jax.ShapeDtypeStruct((N, 4), jnp.float32),
)


def _make_sc_conv(with_z):
    mesh = plsc.VectorSubcoreMesh(core_axis_name="c", subcore_axis_name="s",
                                  num_cores=NC, num_subcores=NS)

    def body(*refs):
        if with_z:
            (u_hbm, z_hbm, c_hbm, src_hbm, dst_hbm, out_hbm,
             srcv0, srcv1, dstv0, dstv1, urows, zrows, crows, acc,
             sem_u, sem_z, sem_c) = refs
        else:
            (u_hbm, c_hbm, src_hbm, dst_hbm, out_hbm,
             srcv0, srcv1, dstv0, dstv1, urows, crows, acc,
             sem_u, sem_c) = refs
        srcvs, dstvs = (srcv0, srcv1), (dstv0, dstv1)
        cid = lax.axis_index("c")
        sid = lax.axis_index("s")
        wid = sid * NC + cid

        def zrow(i, _):
            for j in range(ROWS_F32):
                crows[i, pl.ds(j * 16, 16)] = jnp.zeros((16,), jnp.float32)
            return 0
        lax.fori_loop(0, BLK, zrow, 0)
        row0 = sid * RPT
        off = 0
        for sz in _CHUNKS:
            pltpu.sync_copy(crows.at[pl.ds(0, sz)],
                            acc.at[pl.ds(row0 + off, sz)])
            off += sz
        plsc.subcore_barrier()

        ebase = wid * EPW
        nblk = EPW // BLK
        last_base = ebase + (nblk - 1) * BLK

        def load_idx(p, base):
            pltpu.sync_copy(src_hbm.at[pl.ds(base, BLK)], srcvs[p])
            pltpu.sync_copy(dst_hbm.at[pl.ds(base, BLK)], dstvs[p])

        def do_block(p, base, nxt_base):
            cp_u = pltpu.async_copy(u_hbm.at[srcvs[p]], urows, sem_u)
            if with_z:
                cp_z = pltpu.async_copy(z_hbm.at[dstvs[p]], zrows, sem_z)
            cp_c = pltpu.async_copy(c_hbm.at[pl.ds(base, BLK)], crows, sem_c)
            load_idx(1 - p, nxt_base)
            cp_u.wait()
            if with_z:
                cp_z.wait()
            cp_c.wait()

            def rowb(i, _):
                for j in range(ROWS_F32):
                    sl = pl.ds(j * 16, 16)
                    v = urows[i, sl] + crows[i, sl]
                    if with_z:
                        v = v - zrows[i, sl]
                    urows[i, sl] = jnp.maximum(v, 0.0)
                return 0
            lax.fori_loop(0, BLK, rowb, 0)
            pltpu.sync_copy(urows, acc.at[dstvs[p]], add=True)

        load_idx(0, ebase)

        def pair(i, _):
            b0 = ebase + (2 * i) * BLK
            do_block(0, b0, b0 + BLK)
            do_block(1, b0 + BLK, jnp.minimum(b0 + 2 * BLK, last_base))
            return 0
        lax.fori_loop(0, nblk // 2, pair, 0)

        plsc.subcore_barrier()
        off = 0
        for sz in _CHUNKS:
            r = row0 + off
            pltpu.sync_copy(acc.at[pl.ds(r, sz)], crows.at[pl.ds(0, sz)])
            pltpu.sync_copy(crows.at[pl.ds(0, sz)], out_hbm.at[cid, pl.ds(r, sz)])
            off += sz

    scratch = [pltpu.VMEM((BLK,), jnp.int32)] * 4
    scratch += [pltpu.VMEM((BLK, D), jnp.float32)]
    if with_z:
        scratch += [pltpu.VMEM((BLK, D), jnp.float32)]
    scratch += [
        pltpu.VMEM((BLK, D), jnp.float32),
        pltpu.VMEM_SHARED((N_PAD, D), jnp.float32),
        pltpu.SemaphoreType.DMA,
        pltpu.SemaphoreType.DMA,
    ]
    if with_z:
        scratch += [pltpu.SemaphoreType.DMA]

    return pl.kernel(
        body,
        out_type=jax.ShapeDtypeStruct((2, N_PAD, D), jnp.float32),
        mesh=mesh,
        scratch_types=scratch,
    )


_sc_conv0 = _make_sc_conv(with_z=False)
_sc_conv = _make_sc_conv(with_z=True)


def kernel(x, edge_index, edge_attr, W_enc, b_enc, W_dec, b_dec, W_msg,
           W_edge, b_msg, W_self):
    src = edge_index[0]
    dst = edge_index[1]
    pad = E_PAD - E
    src_p = jnp.concatenate([src, jnp.zeros((pad,), jnp.int32)])
    dst_p = jnp.concatenate([dst, jnp.full((pad,), N, jnp.int32)])
    ea_p = jnp.concatenate([edge_attr, jnp.zeros((pad, 16), jnp.float32)])

    row_ids = lax.broadcasted_iota(jnp.int32, (16, 1), 0)
    keep = jnp.logical_or(row_ids < 2, row_ids >= 5).astype(jnp.float32)
    W_edge_z = W_edge * keep
    W3 = W_edge[2:5]
    Wz = W_dec[:, :3] @ W3
    bz = (b_dec[:3] @ W3).reshape(1, D)
    b_msg2 = b_msg.reshape(1, D)
    b_enc2 = b_enc.reshape(1, D)
    b_dec2 = b_dec.reshape(1, 4)

    W_self_i = W_self + jnp.eye(D, dtype=jnp.float32)

    c0, cz = _prep_call(ea_p, W_edge, W_edge_z, b_msg2)

    h, u = _enc_call(x, W_enc, b_enc2, W_msg)
    agg = _sc_conv0(u, c0, src_p, dst_p)
    h, u, _z = _mid_call(h, agg[0, :N], agg[1, :N], W_self_i, W_msg, Wz, bz)
    agg = _sc_conv(u, _z, cz, src_p, dst_p)
    h, u, _z = _mid_call(h, agg[0, :N], agg[1, :N], W_self_i, W_msg, Wz, bz)
    agg = _sc_conv(u, _z, cz, src_p, dst_p)
    out = _fin_call(h, agg[0, :N], agg[1, :N], W_self_i, W_dec, b_dec2)
    return out


# speedup vs baseline: 4.8084x; 1.2024x over previous
import functools

import jax
import jax.numpy as jnp
from jax import lax
from jax.experimental import pallas as pl
from jax.experimental.pallas import tpu as pltpu
from jax.experimental.pallas import tpu_sc as plsc

N = 10000
D = 128
E = 320000
NC = 2
NS = 16
NW = NC * NS
BLK = 64
EPW = 10240
E_PAD = NW * EPW
N_PAD = 10112
RPT = N_PAD // NS
ROWS_F32 = (D // 16)
_CHUNKS = [BLK] * (RPT // BLK) + ([RPT % BLK] if RPT % BLK else [])


def _prep_body(ea_ref, we_ref, wez_ref, b_ref, c0_ref, cz_ref):
    ea = ea_ref[...]
    b = b_ref[...]
    c0_ref[...] = jnp.dot(ea, we_ref[...], preferred_element_type=jnp.float32) + b
    cz_ref[...] = jnp.dot(ea, wez_ref[...], preferred_element_type=jnp.float32) + b


def _enc_body(x_ref, wenc_ref, benc_ref, wmsg_ref, h_ref, u_ref):
    h = jnp.dot(x_ref[...], wenc_ref[...], preferred_element_type=jnp.float32)
    h = h + benc_ref[...]
    h_ref[...] = h
    u_ref[...] = jnp.dot(h, wmsg_ref[...], preferred_element_type=jnp.float32)


def _mid_body(h_ref, agg0_ref, agg1_ref, wself_ref, wmsg_ref, wz_ref, bz_ref,
              h_out, u_out, z_out):
    h = jnp.dot(h_ref[...], wself_ref[...], preferred_element_type=jnp.float32)
    h = h + agg0_ref[...] + agg1_ref[...]
    z = jnp.dot(h, wz_ref[...], preferred_element_type=jnp.float32) + bz_ref[...]
    h_out[...] = h
    z_out[...] = z
    u_out[...] = jnp.dot(h, wmsg_ref[...], preferred_element_type=jnp.float32) + z


def _fin_body(h_ref, agg0_ref, agg1_ref, wself_ref, wdec_ref, bdec_ref, out_ref):
    h = jnp.dot(h_ref[...], wself_ref[...], preferred_element_type=jnp.float32)
    h = h + agg0_ref[...] + agg1_ref[...]
    out_ref[...] = jnp.dot(h, wdec_ref[...], preferred_element_type=jnp.float32)
    out_ref[...] += bdec_ref[...]


_W_SPEC = pl.BlockSpec((16, D), lambda i: (0, 0))
_WD_SPEC = pl.BlockSpec((D, D), lambda i: (0, 0))
_B_SPEC = pl.BlockSpec((1, D), lambda i: (0, 0))

_NBLK_ROWS = 2000
_ROW_SPEC = pl.BlockSpec((_NBLK_ROWS, D), lambda i: (i, 0))

_EBLK = 4096
_prep_call = pl.pallas_call(
    _prep_body,
    grid=(E_PAD // _EBLK,),
    in_specs=[pl.BlockSpec((_EBLK, 16), lambda i: (i, 0)), _W_SPEC, _W_SPEC,
              _B_SPEC],
    out_specs=[pl.BlockSpec((_EBLK, D), lambda i: (i, 0))] * 2,
    out_shape=[jax.ShapeDtypeStruct((E_PAD, D), jnp.float32)] * 2,
)

_enc_call = pl.pallas_call(
    _enc_body,
    grid=(N // _NBLK_ROWS,),
    in_specs=[_ROW_SPEC, _WD_SPEC, _B_SPEC, _WD_SPEC],
    out_specs=[_ROW_SPEC, _ROW_SPEC],
    out_shape=[jax.ShapeDtypeStruct((N, D), jnp.float32)] * 2,
)

_mid_call = pl.pallas_call(
    _mid_body,
    grid=(N // _NBLK_ROWS,),
    in_specs=[_ROW_SPEC, _ROW_SPEC, _ROW_SPEC, _WD_SPEC, _WD_SPEC, _WD_SPEC,
              _B_SPEC],
    out_specs=[_ROW_SPEC, _ROW_SPEC, _ROW_SPEC],
    out_shape=[jax.ShapeDtypeStruct((N, D), jnp.float32)] * 3,
)

_fin_call = pl.pallas_call(
    _fin_body,
    grid=(N // _NBLK_ROWS,),
    in_specs=[_ROW_SPEC, _ROW_SPEC, _ROW_SPEC, _WD_SPEC,
              pl.BlockSpec((D, 4), lambda i: (0, 0)),
              pl.BlockSpec((1, 4), lambda i: (0, 0))],
    out_specs=pl.BlockSpec((_NBLK_ROWS, 4), lambda i: (i, 0)),
    out_shape=jax.ShapeDtypeStruct((N, 4), jnp.float32),
)


def _make_sc_conv(with_z):
    mesh = plsc.VectorSubcoreMesh(core_axis_name="c", subcore_axis_name="s",
                                  num_cores=NC, num_subcores=NS)

    def body(*refs):
        if with_z:
            (u_hbm, z_hbm, c_hbm, src_hbm, dst_hbm, out_hbm,
             srcv0, srcv1, dstv0, dstv1, u0, u1, z0, z1, cb0, cb1, acc,
             su0, su1, sz0, sz1, sc0, sc1) = refs
            zbufs, zsems = (z0, z1), (sz0, sz1)
        else:
            (u_hbm, c_hbm, src_hbm, dst_hbm, out_hbm,
             srcv0, srcv1, dstv0, dstv1, u0, u1, cb0, cb1, acc,
             su0, su1, sc0, sc1) = refs
        srcvs, dstvs = (srcv0, srcv1), (dstv0, dstv1)
        ubufs, cbufs = (u0, u1), (cb0, cb1)
        usems, csems = (su0, su1), (sc0, sc1)
        cid = lax.axis_index("c")
        sid = lax.axis_index("s")
        wid = sid * NC + cid

        def zrow(i, _):
            for j in range(ROWS_F32):
                cb0[i, pl.ds(j * 16, 16)] = jnp.zeros((16,), jnp.float32)
            return 0
        lax.fori_loop(0, BLK, zrow, 0)
        row0 = sid * RPT
        off = 0
        for sz in _CHUNKS:
            pltpu.sync_copy(cb0.at[pl.ds(0, sz)],
                            acc.at[pl.ds(row0 + off, sz)])
            off += sz
        plsc.subcore_barrier()

        ebase = wid * EPW
        nblk = EPW // BLK
        last_base = ebase + (nblk - 1) * BLK

        def load_idx(p, base):
            pltpu.sync_copy(src_hbm.at[pl.ds(base, BLK)], srcvs[p])
            pltpu.sync_copy(dst_hbm.at[pl.ds(base, BLK)], dstvs[p])

        def start(p, base):
            pltpu.async_copy(u_hbm.at[srcvs[p]], ubufs[p], usems[p])
            if with_z:
                pltpu.async_copy(z_hbm.at[dstvs[p]], zbufs[p], zsems[p])
            pltpu.async_copy(c_hbm.at[pl.ds(base, BLK)], cbufs[p], csems[p])

        def wait(p, base):
            pltpu.make_async_copy(u_hbm.at[srcvs[p]], ubufs[p],
                                  usems[p]).wait()
            if with_z:
                pltpu.make_async_copy(z_hbm.at[dstvs[p]], zbufs[p],
                                      zsems[p]).wait()
            pltpu.make_async_copy(c_hbm.at[pl.ds(base, BLK)], cbufs[p],
                                  csems[p]).wait()

        def do_slot(p, base):
            wait(p, base)
            ub, cb = ubufs[p], cbufs[p]
            if with_z:
                zb = zbufs[p]

            def rowb(i, _):
                for j in range(ROWS_F32):
                    sl = pl.ds(j * 16, 16)
                    v = ub[i, sl] + cb[i, sl]
                    if with_z:
                        v = v - zb[i, sl]
                    ub[i, sl] = jnp.maximum(v, 0.0)
                return 0
            lax.fori_loop(0, BLK, rowb, 0)
            pltpu.sync_copy(ub, acc.at[dstvs[p]], add=True)
            nxt = jnp.minimum(base + 2 * BLK, last_base)
            load_idx(p, nxt)
            start(p, nxt)

        load_idx(0, ebase)
        start(0, ebase)
        load_idx(1, ebase + BLK)
        start(1, ebase + BLK)

        def pair(i, _):
            b0 = ebase + (2 * i) * BLK
            do_slot(0, b0)
            do_slot(1, b0 + BLK)
            return 0
        lax.fori_loop(0, nblk // 2, pair, 0)
        wait(0, last_base)
        wait(1, last_base)

        plsc.subcore_barrier()
        off = 0
        for sz in _CHUNKS:
            r = row0 + off
            pltpu.sync_copy(acc.at[pl.ds(r, sz)], cb0.at[pl.ds(0, sz)])
            pltpu.sync_copy(cb0.at[pl.ds(0, sz)], out_hbm.at[cid, pl.ds(r, sz)])
            off += sz

    scratch = [pltpu.VMEM((BLK,), jnp.int32)] * 4
    scratch += [pltpu.VMEM((BLK, D), jnp.float32)] * 2
    if with_z:
        scratch += [pltpu.VMEM((BLK, D), jnp.float32)] * 2
    scratch += [
        pltpu.VMEM((BLK, D), jnp.float32),
        pltpu.VMEM((BLK, D), jnp.float32),
        pltpu.VMEM_SHARED((N_PAD, D), jnp.float32),
    ]
    scratch += [pltpu.SemaphoreType.DMA] * (6 if with_z else 4)

    return pl.kernel(
        body,
        out_type=jax.ShapeDtypeStruct((2, N_PAD, D), jnp.float32),
        mesh=mesh,
        scratch_types=scratch,
    )


_sc_conv0 = _make_sc_conv(with_z=False)
_sc_conv = _make_sc_conv(with_z=True)


def kernel(x, edge_index, edge_attr, W_enc, b_enc, W_dec, b_dec, W_msg,
           W_edge, b_msg, W_self):
    src = edge_index[0]
    dst = edge_index[1]
    pad = E_PAD - E
    src_p = jnp.concatenate([src, jnp.zeros((pad,), jnp.int32)])
    dst_p = jnp.concatenate([dst, jnp.full((pad,), N, jnp.int32)])
    ea_p = jnp.concatenate([edge_attr, jnp.zeros((pad, 16), jnp.float32)])

    row_ids = lax.broadcasted_iota(jnp.int32, (16, 1), 0)
    keep = jnp.logical_or(row_ids < 2, row_ids >= 5).astype(jnp.float32)
    W_edge_z = W_edge * keep
    W3 = W_edge[2:5]
    Wz = W_dec[:, :3] @ W3
    bz = (b_dec[:3] @ W3).reshape(1, D)
    b_msg2 = b_msg.reshape(1, D)
    b_enc2 = b_enc.reshape(1, D)
    b_dec2 = b_dec.reshape(1, 4)

    W_self_i = W_self + jnp.eye(D, dtype=jnp.float32)

    c0, cz = _prep_call(ea_p, W_edge, W_edge_z, b_msg2)

    h, u = _enc_call(x, W_enc, b_enc2, W_msg)
    agg = _sc_conv0(u, c0, src_p, dst_p)
    h, u, _z = _mid_call(h, agg[0, :N], agg[1, :N], W_self_i, W_msg, Wz, bz)
    agg = _sc_conv(u, _z, cz, src_p, dst_p)
    h, u, _z = _mid_call(h, agg[0, :N], agg[1, :N], W_self_i, W_msg, Wz, bz)
    agg = _sc_conv(u, _z, cz, src_p, dst_p)
    out = _fin_call(h, agg[0, :N], agg[1, :N], W_self_i, W_dec, b_dec2)
    return out
